# baseline (speedup 1.0000x reference)
import functools

import jax
import jax.numpy as jnp
from jax import lax
from jax.experimental import pallas as pl
from jax.experimental.pallas import tpu as pltpu
from jax.experimental.pallas import tpu_sc as plsc

N = 50000
E = 800000
R = 4
F = 16
NH = N // 2
SEGS = R * NH
ROWS = SEGS + 96
CROWS = SEGS + 352
ZR = ROWS // 16
CZ = CROWS // 16
ZLINE = CZ // 4
OUT_CH = 1000
NOUT = SEGS // OUT_CH

_mesh = plsc.VectorSubcoreMesh(core_axis_name="c", subcore_axis_name="s")


def _make_sc_scatter(with_counts, nx):
    Cc = 400 if with_counts else 640
    nchunk = E // Cc
    nouter = (-(-nchunk // 16) + 1) // 2

    out_type = [jax.ShapeDtypeStruct((R * N, F), jnp.float32)
                for _ in range(nx)]
    scratch = [
        pltpu.VMEM_SHARED((ROWS, F), jnp.float32),
        pltpu.VMEM((Cc,), jnp.int32),
        pltpu.VMEM((Cc,), jnp.int32),
        pltpu.VMEM((Cc,), jnp.int32),
        pltpu.VMEM((Cc,), jnp.int32),
        pltpu.VMEM((Cc,), jnp.int32),
        pltpu.VMEM((Cc,), jnp.int32),
        pltpu.VMEM((Cc,), jnp.int32),
        pltpu.VMEM((Cc,), jnp.int32),
        pltpu.VMEM((Cc, F), jnp.float32),
        pltpu.VMEM((Cc, F), jnp.float32),
        pltpu.SemaphoreType.DMA,
        pltpu.SemaphoreType.DMA,
        pltpu.SemaphoreType.DMA,
        pltpu.SemaphoreType.DMA,
        pltpu.SemaphoreType.DMA,
        pltpu.SemaphoreType.DMA,
    ]
    if with_counts:
        out_type.append(jax.ShapeDtypeStruct((R * N,), jnp.float32))
        scratch += [
            pltpu.VMEM_SHARED((CROWS,), jnp.float32),
            pltpu.VMEM((Cc,), jnp.int32),
            pltpu.VMEM((Cc,), jnp.int32),
            pltpu.VMEM((Cc,), jnp.float32),
            pltpu.VMEM((ZLINE,), jnp.float32),
            pltpu.SemaphoreType.DMA,
            pltpu.SemaphoreType.DMA,
        ]

    def body(*refs):
        xs = refs[:nx]
        ei_hbm, et_hbm = refs[nx:nx + 2]
        aggrs = refs[nx + 2:2 * nx + 2]
        rest = refs[2 * nx + 2:]
        if with_counts:
            (cnt_hbm, shared, src0, src1, dst0, dst1, typ0, typ1,
             seg0, seg1, xb0, xb1, is0, is1, gs0, gs1, ss0, ss1,
             cshared, sgc0, sgc1, onesv, zflat, cs0, cs1) = rest
            segcv = (sgc0, sgc1)
            csem = (cs0, cs1)
        else:
            (shared, src0, src1, dst0, dst1, typ0, typ1,
             seg0, seg1, xb0, xb1, is0, is1, gs0, gs1, ss0, ss1) = rest
        srcv = (src0, src1)
        dstv = (dst0, dst1)
        typv = (typ0, typ1)
        segv = (seg0, seg1)
        xbuf = (xb0, xb1)
        isem = (is0, is1)
        gsem = (gs0, gs1)
        ssem = (ss0, ss1)
        cc = lax.axis_index("c")
        s = lax.axis_index("s")
        base = cc * NH
        dump = SEGS + s

        def idx_copies(cid, p):
            e0 = cid * Cc
            return (
                pltpu.make_async_copy(ei_hbm.at[0, pl.ds(e0, Cc)],
                                      srcv[p], isem[p]),
                pltpu.make_async_copy(ei_hbm.at[1, pl.ds(e0, Cc)],
                                      dstv[p], isem[p]),
                pltpu.make_async_copy(et_hbm.at[pl.ds(e0, Cc)],
                                      typv[p], isem[p]),
            )

        def idx_start(cid, p):
            for d in idx_copies(cid, p):
                d.start()

        def idx_wait(cid, p):
            for d in idx_copies(cid, p):
                d.wait()

        def zero_accumulators(first):
            def zb(i, _):
                xb0[i, :] = jnp.zeros((F,), jnp.float32)
                return 0
            lax.fori_loop(0, Cc, zb, 0)
            off = s * ZR
            for kk in range(ZR // Cc):
                pltpu.sync_copy(xb0, shared.at[pl.ds(off + kk * Cc, Cc)])
            rem = ZR % Cc
            if rem:
                pltpu.sync_copy(xb0.at[pl.ds(0, rem)],
                                shared.at[pl.ds(off + (ZR // Cc) * Cc, rem)])
            if with_counts and first:
                def zf(i, _):
                    zflat[pl.ds(i * 16, 16)] = jnp.zeros((16,), jnp.float32)
                    return 0
                lax.fori_loop(0, ZLINE // 16, zf, 0)
                for kk in range(4):
                    pltpu.sync_copy(
                        zflat, cshared.at[pl.ds(s * CZ + kk * ZLINE, ZLINE)])

                def ob(i, _):
                    onesv[pl.ds(i * 16, 16)] = jnp.ones((16,), jnp.float32)
                    return 0
                lax.fori_loop(0, Cc // 16, ob, 0)

        def scat_desc(p):
            return pltpu.make_async_copy(xbuf[p], shared.at[segv[p]],
                                         ssem[p])

        def cscat_desc(p):
            return pltpu.make_async_copy(onesv, cshared.at[segcv[p]],
                                         csem[p])

        def sweep(x_hbm, counts):
            def outer(i, _):
                for p in (0, 1):
                    k = 2 * i + p
                    cid = k * 16 + s

                    @pl.when(cid < nchunk)
                    def _():
                        @pl.when(i >= 1)
                        def _():
                            scat_desc(p).wait()
                            if counts:
                                cscat_desc(p).wait()
                        idx_wait(cid, p)
                        g = pltpu.async_copy(x_hbm.at[srcv[p]], xbuf[p],
                                             gsem[p])

                        def segb(j, _2):
                            d = dstv[p][pl.ds(j * 16, 16)]
                            t = typv[p][pl.ds(j * 16, 16)]
                            loc = d - base
                            ok = (loc >= 0) & (loc < NH)
                            segv[p][pl.ds(j * 16, 16)] = jnp.where(
                                ok, t * NH + loc, dump)
                            if counts:
                                segcv[p][pl.ds(j * 16, 16)] = jnp.where(
                                    ok, loc * R + t, dump)
                            return 0
                        lax.fori_loop(0, Cc // 16, segb, 0)
                        g.wait()

                        @pl.when(cid + 32 < nchunk)
                        def _():
                            idx_start(cid + 32, p)
                        pltpu.async_copy(xbuf[p], shared.at[segv[p]],
                                         ssem[p], add=True)
                        if counts:
                            pltpu.async_copy(onesv, cshared.at[segcv[p]],
                                             csem[p], add=True)
                return 0
            lax.fori_loop(0, nouter, outer, 0)

            ntile = (nchunk - s + 15) // 16
            for p in (0, 1):
                @pl.when(ntile >= p + 1)
                def _():
                    scat_desc(p).wait()
                    if counts:
                        cscat_desc(p).wait()

        def copy_out(aggr_hbm):
            def cpout(k, _):
                cid = k * 16 + s

                @pl.when(cid < NOUT)
                def _():
                    r = cid // (NOUT // R)
                    i0 = (cid % (NOUT // R)) * OUT_CH
                    pltpu.sync_copy(
                        shared.at[pl.ds(r * NH + i0, OUT_CH)],
                        aggr_hbm.at[pl.ds(r * N + base + i0, OUT_CH)])
                return 0
            lax.fori_loop(0, -(-NOUT // 16), cpout, 0)

        for xi in range(nx):
            counts = with_counts and xi == 0
            idx_start(s, 0)
            idx_start(16 + s, 1)
            zero_accumulators(first=xi == 0)
            plsc.subcore_barrier()
            sweep(xs[xi], counts)
            plsc.subcore_barrier()
            copy_out(aggrs[xi])
            if counts:
                def cpc(k, _):
                    cid = k * 16 + s

                    @pl.when(cid < NOUT)
                    def _():
                        pltpu.sync_copy(
                            cshared.at[pl.ds(cid * OUT_CH, OUT_CH)],
                            cnt_hbm.at[pl.ds(cc * SEGS + cid * OUT_CH,
                                             OUT_CH)])
                    return 0
                lax.fori_loop(0, -(-NOUT // 16), cpc, 0)
            if xi + 1 < nx:
                plsc.subcore_barrier()

    return pl.kernel(body, out_type=tuple(out_type) if len(out_type) > 1
                     else out_type[0],
                     mesh=_mesh, scratch_types=scratch,
                     compiler_params=pltpu.CompilerParams(
                         use_tc_tiling_on_sc=False))


_sc_scatter_counts = _make_sc_scatter(True, 1)
_sc_scatter = _make_sc_scatter(False, 1)
_sc_scatter2 = _make_sc_scatter(False, 2)


def _dense(h_parts, aggrs, cnt2, root, W, b, fout, relu, split):
    B = 2000
    P = len(h_parts)
    fin = P * F
    in_specs = (
        [pl.BlockSpec((B, F), lambda i: (i, 0)) for _ in range(P)]
        + [pl.BlockSpec((R, B, F), lambda i: (0, i, 0)) for _ in range(P)]
        + [pl.BlockSpec((B, R), lambda i: (i, 0)),
           pl.BlockSpec((fin, fout), lambda i: (0, 0)),
           pl.BlockSpec((R, fin, fout), lambda i: (0, 0, 0)),
           pl.BlockSpec((1, fout), lambda i: (0, 0))]
    )
    if split:
        out_shape = [jax.ShapeDtypeStruct((N, F), jnp.float32)] * 2
        out_specs = [pl.BlockSpec((B, F), lambda i: (i, 0))] * 2
    else:
        out_shape = jax.ShapeDtypeStruct((N, fout), jnp.float32)
        out_specs = pl.BlockSpec((B, fout), lambda i: (i, 0))

    def body(*refs):
        hs = refs[:P]
        ags = refs[P:2 * P]
        cref, rref, wref, bref = refs[2 * P:2 * P + 4]
        outs = refs[2 * P + 4:]
        inv = 1.0 / jnp.maximum(cref[...], 1.0)
        parts = [h[...] for h in hs]
        for aref in ags:
            a = aref[...]
            for r in range(R):
                parts.append(a[r] * inv[:, r:r + 1])
        xcat = jnp.concatenate(parts, axis=-1)
        rv = rref[...]
        wv = wref[...]
        wparts = [rv[p * F:(p + 1) * F] for p in range(P)]
        for p in range(P):
            for r in range(R):
                wparts.append(wv[r][p * F:(p + 1) * F])
        wcat = jnp.concatenate(wparts, axis=0)
        y = jnp.dot(xcat, wcat, preferred_element_type=jnp.float32)
        y = y + bref[...]
        if relu:
            y = jnp.maximum(y, 0.0)
        if split:
            outs[0][...] = y[:, :F]
            outs[1][...] = y[:, F:]
        else:
            outs[0][...] = y

    return pl.pallas_call(body, grid=(N // B,), in_specs=in_specs,
                          out_specs=out_specs, out_shape=out_shape)(
        *h_parts, *aggrs, cnt2, root, W, b)


def kernel(x, edge_index, edge_type, W1, root1, b1, W2, root2, b2, W3, root3, b3):
    ei = edge_index
    et = edge_type

    aggr1f, cntf = _sc_scatter_counts(x, ei, et)
    aggr1 = aggr1f.reshape(R, N, F)
    cnt2 = cntf.reshape(N, R)

    h2 = _dense([x], [aggr1], cnt2, root1, W1, b1.reshape(1, -1), 16,
                relu=True, split=False)

    aggr2 = _sc_scatter(h2, ei, et).reshape(R, N, F)
    h3a, h3b = _dense([h2], [aggr2], cnt2, root2, W2, b2.reshape(1, -1), 32,
                      relu=True, split=True)

    aggr3af, aggr3bf = _sc_scatter2(h3a, h3b, ei, et)
    out = _dense([h3a, h3b],
                 [aggr3af.reshape(R, N, F), aggr3bf.reshape(R, N, F)],
                 cnt2, root3, W3, b3.reshape(1, -1), 64,
                 relu=False, split=False)
    return out


# speedup vs baseline: 1.1225x; 1.1225x over previous
import functools

import jax
import jax.numpy as jnp
from jax import lax
from jax.experimental import pallas as pl
from jax.experimental.pallas import tpu as pltpu
from jax.experimental.pallas import tpu_sc as plsc

N = 50000
E = 800000
R = 4
F = 16
NH = N // 2
SEGS = R * NH
ROWS = SEGS + 96
CROWS = SEGS + 352
ZR = ROWS // 16
CZ = CROWS // 16
ZLINE = CZ // 4
OUT_CH = 1000
NOUT = SEGS // OUT_CH

_mesh = plsc.VectorSubcoreMesh(core_axis_name="c", subcore_axis_name="s")


def _make_sc_scatter(with_counts, nx):
    Cc = 400 if with_counts else 640
    nchunk = E // Cc
    nouter = (-(-nchunk // 16) + 1) // 2

    out_type = [jax.ShapeDtypeStruct((N, R * F), jnp.float32)
                for _ in range(nx)]
    scratch = [
        pltpu.VMEM_SHARED((ROWS, F), jnp.float32),
        pltpu.VMEM((Cc,), jnp.int32),
        pltpu.VMEM((Cc,), jnp.int32),
        pltpu.VMEM((Cc,), jnp.int32),
        pltpu.VMEM((Cc,), jnp.int32),
        pltpu.VMEM((Cc,), jnp.int32),
        pltpu.VMEM((Cc,), jnp.int32),
        pltpu.VMEM((Cc,), jnp.int32),
        pltpu.VMEM((Cc,), jnp.int32),
        pltpu.VMEM((Cc, F), jnp.float32),
        pltpu.VMEM((Cc, F), jnp.float32),
        pltpu.SemaphoreType.DMA,
        pltpu.SemaphoreType.DMA,
        pltpu.SemaphoreType.DMA,
        pltpu.SemaphoreType.DMA,
        pltpu.SemaphoreType.DMA,
        pltpu.SemaphoreType.DMA,
    ]
    if with_counts:
        out_type.append(jax.ShapeDtypeStruct((R * N,), jnp.float32))
        scratch += [
            pltpu.VMEM_SHARED((CROWS,), jnp.float32),
            pltpu.VMEM((Cc,), jnp.int32),
            pltpu.VMEM((Cc,), jnp.int32),
            pltpu.VMEM((Cc,), jnp.float32),
            pltpu.VMEM((ZLINE,), jnp.float32),
            pltpu.SemaphoreType.DMA,
            pltpu.SemaphoreType.DMA,
        ]

    def body(*refs):
        xs = refs[:nx]
        ei_hbm, et_hbm = refs[nx:nx + 2]
        aggrs = refs[nx + 2:2 * nx + 2]
        rest = refs[2 * nx + 2:]
        if with_counts:
            (cnt_hbm, shared, src0, src1, dst0, dst1, typ0, typ1,
             seg0, seg1, xb0, xb1, is0, is1, gs0, gs1, ss0, ss1,
             cshared, sgc0, sgc1, onesv, zflat, cs0, cs1) = rest
            segcv = (sgc0, sgc1)
            csem = (cs0, cs1)
        else:
            (shared, src0, src1, dst0, dst1, typ0, typ1,
             seg0, seg1, xb0, xb1, is0, is1, gs0, gs1, ss0, ss1) = rest
        srcv = (src0, src1)
        dstv = (dst0, dst1)
        typv = (typ0, typ1)
        segv = (seg0, seg1)
        xbuf = (xb0, xb1)
        isem = (is0, is1)
        gsem = (gs0, gs1)
        ssem = (ss0, ss1)
        cc = lax.axis_index("c")
        s = lax.axis_index("s")
        base = cc * NH
        dump = SEGS + s

        def idx_copies(cid, p):
            e0 = cid * Cc
            return (
                pltpu.make_async_copy(ei_hbm.at[0, pl.ds(e0, Cc)],
                                      srcv[p], isem[p]),
                pltpu.make_async_copy(ei_hbm.at[1, pl.ds(e0, Cc)],
                                      dstv[p], isem[p]),
                pltpu.make_async_copy(et_hbm.at[pl.ds(e0, Cc)],
                                      typv[p], isem[p]),
            )

        def idx_start(cid, p):
            for d in idx_copies(cid, p):
                d.start()

        def idx_wait(cid, p):
            for d in idx_copies(cid, p):
                d.wait()

        def zero_accumulators(first):
            def zb(i, _):
                xb0[i, :] = jnp.zeros((F,), jnp.float32)
                return 0
            lax.fori_loop(0, Cc, zb, 0)
            off = s * ZR
            for kk in range(ZR // Cc):
                pltpu.sync_copy(xb0, shared.at[pl.ds(off + kk * Cc, Cc)])
            rem = ZR % Cc
            if rem:
                pltpu.sync_copy(xb0.at[pl.ds(0, rem)],
                                shared.at[pl.ds(off + (ZR // Cc) * Cc, rem)])
            if with_counts and first:
                def zf(i, _):
                    zflat[pl.ds(i * 16, 16)] = jnp.zeros((16,), jnp.float32)
                    return 0
                lax.fori_loop(0, ZLINE // 16, zf, 0)
                for kk in range(4):
                    pltpu.sync_copy(
                        zflat, cshared.at[pl.ds(s * CZ + kk * ZLINE, ZLINE)])

                def ob(i, _):
                    onesv[pl.ds(i * 16, 16)] = jnp.ones((16,), jnp.float32)
                    return 0
                lax.fori_loop(0, Cc // 16, ob, 0)

        def scat_desc(p):
            return pltpu.make_async_copy(xbuf[p], shared.at[segv[p]],
                                         ssem[p])

        def cscat_desc(p):
            return pltpu.make_async_copy(onesv, cshared.at[segcv[p]],
                                         csem[p])

        def sweep(x_hbm, counts):
            def outer(i, _):
                for p in (0, 1):
                    k = 2 * i + p
                    cid = k * 16 + s

                    @pl.when(cid < nchunk)
                    def _():
                        @pl.when(i >= 1)
                        def _():
                            scat_desc(p).wait()
                            if counts:
                                cscat_desc(p).wait()
                        idx_wait(cid, p)
                        g = pltpu.async_copy(x_hbm.at[srcv[p]], xbuf[p],
                                             gsem[p])

                        def segb(j, _2):
                            d = dstv[p][pl.ds(j * 16, 16)]
                            t = typv[p][pl.ds(j * 16, 16)]
                            loc = d - base
                            ok = (loc >= 0) & (loc < NH)
                            segv[p][pl.ds(j * 16, 16)] = jnp.where(
                                ok, t * NH + loc, dump)
                            if counts:
                                segcv[p][pl.ds(j * 16, 16)] = jnp.where(
                                    ok, loc * R + t, dump)
                            return 0
                        lax.fori_loop(0, Cc // 16, segb, 0)
                        g.wait()

                        @pl.when(cid + 32 < nchunk)
                        def _():
                            idx_start(cid + 32, p)
                        pltpu.async_copy(xbuf[p], shared.at[segv[p]],
                                         ssem[p], add=True)
                        if counts:
                            pltpu.async_copy(onesv, cshared.at[segcv[p]],
                                             csem[p], add=True)
                return 0
            lax.fori_loop(0, nouter, outer, 0)

            ntile = (nchunk - s + 15) // 16
            for p in (0, 1):
                @pl.when(ntile >= p + 1)
                def _():
                    scat_desc(p).wait()
                    if counts:
                        cscat_desc(p).wait()

        def copy_out(aggr_hbm):
            def cpout(k, _):
                cid = k * 16 + s

                @pl.when(cid < NOUT)
                def _():
                    r = cid // (NOUT // R)
                    i0 = (cid % (NOUT // R)) * OUT_CH
                    pltpu.sync_copy(
                        shared.at[pl.ds(r * NH + i0, OUT_CH)],
                        aggr_hbm.at[pl.ds(base + i0, OUT_CH),
                                    pl.ds(r * F, F)])
                return 0
            lax.fori_loop(0, -(-NOUT // 16), cpout, 0)

        for xi in range(nx):
            counts = with_counts and xi == 0
            idx_start(s, 0)
            idx_start(16 + s, 1)
            zero_accumulators(first=xi == 0)
            plsc.subcore_barrier()
            sweep(xs[xi], counts)
            plsc.subcore_barrier()
            copy_out(aggrs[xi])
            if counts:
                def cpc(k, _):
                    cid = k * 16 + s

                    @pl.when(cid < NOUT)
                    def _():
                        pltpu.sync_copy(
                            cshared.at[pl.ds(cid * OUT_CH, OUT_CH)],
                            cnt_hbm.at[pl.ds(cc * SEGS + cid * OUT_CH,
                                             OUT_CH)])
                    return 0
                lax.fori_loop(0, -(-NOUT // 16), cpc, 0)
            if xi + 1 < nx:
                plsc.subcore_barrier()

    return pl.kernel(body, out_type=tuple(out_type) if len(out_type) > 1
                     else out_type[0],
                     mesh=_mesh, scratch_types=scratch,
                     compiler_params=pltpu.CompilerParams(
                         use_tc_tiling_on_sc=False))


_sc_scatter_counts = _make_sc_scatter(True, 1)
_sc_scatter = _make_sc_scatter(False, 1)
_sc_scatter2 = _make_sc_scatter(False, 2)


def _dense(h_parts, aggrs, cnt2, root, W, b, fout, relu, split):
    B = 2000
    P = len(h_parts)
    fin = P * F
    in_specs = (
        [pl.BlockSpec((B, F), lambda i: (i, 0)) for _ in range(P)]
        + [pl.BlockSpec((B, R * F), lambda i: (i, 0)) for _ in range(P)]
        + [pl.BlockSpec((B, R), lambda i: (i, 0)),
           pl.BlockSpec((fin, fout), lambda i: (0, 0)),
           pl.BlockSpec((R, fin, fout), lambda i: (0, 0, 0)),
           pl.BlockSpec((1, fout), lambda i: (0, 0))]
    )
    if split:
        out_shape = [jax.ShapeDtypeStruct((N, F), jnp.float32)] * 2
        out_specs = [pl.BlockSpec((B, F), lambda i: (i, 0))] * 2
    else:
        out_shape = jax.ShapeDtypeStruct((N, fout), jnp.float32)
        out_specs = pl.BlockSpec((B, fout), lambda i: (i, 0))

    def body(*refs):
        hs = refs[:P]
        ags = refs[P:2 * P]
        cref, rref, wref, bref = refs[2 * P:2 * P + 4]
        outs = refs[2 * P + 4:]
        inv = 1.0 / jnp.maximum(cref[...], 1.0)
        sel = jnp.repeat(jnp.eye(R, dtype=jnp.float32), F, axis=1)
        scale = jnp.dot(inv, sel, preferred_element_type=jnp.float32)
        parts = [h[...] for h in hs]
        for aref in ags:
            parts.append(aref[...] * scale)
        xcat = jnp.concatenate(parts, axis=-1)
        rv = rref[...]
        wv = wref[...]
        wparts = [rv[p * F:(p + 1) * F] for p in range(P)]
        for p in range(P):
            for r in range(R):
                wparts.append(wv[r][p * F:(p + 1) * F])
        wcat = jnp.concatenate(wparts, axis=0)
        y = jnp.dot(xcat, wcat, preferred_element_type=jnp.float32)
        y = y + bref[...]
        if relu:
            y = jnp.maximum(y, 0.0)
        if split:
            outs[0][...] = y[:, :F]
            outs[1][...] = y[:, F:]
        else:
            outs[0][...] = y

    return pl.pallas_call(body, grid=(N // B,), in_specs=in_specs,
                          out_specs=out_specs, out_shape=out_shape)(
        *h_parts, *aggrs, cnt2, root, W, b)


def kernel(x, edge_index, edge_type, W1, root1, b1, W2, root2, b2, W3, root3, b3):
    ei = edge_index
    et = edge_type

    aggr1, cntf = _sc_scatter_counts(x, ei, et)
    cnt2 = cntf.reshape(N, R)

    h2 = _dense([x], [aggr1], cnt2, root1, W1, b1.reshape(1, -1), 16,
                relu=True, split=False)

    aggr2 = _sc_scatter(h2, ei, et)
    h3a, h3b = _dense([h2], [aggr2], cnt2, root2, W2, b2.reshape(1, -1), 32,
                      relu=True, split=True)

    aggr3a, aggr3b = _sc_scatter2(h3a, h3b, ei, et)
    out = _dense([h3a, h3b], [aggr3a, aggr3b],
                 cnt2, root3, W3, b3.reshape(1, -1), 64,
                 relu=False, split=False)
    return out


# speedup vs baseline: 1.1295x; 1.0062x over previous
import functools

import jax
import jax.numpy as jnp
from jax import lax
from jax.experimental import pallas as pl
from jax.experimental.pallas import tpu as pltpu
from jax.experimental.pallas import tpu_sc as plsc

N = 50000
E = 800000
R = 4
F = 16
NH = N // 2
SEGS = R * NH
ROWS = SEGS + 96
CROWS = SEGS + 352
ZR = ROWS // 16
CZ = CROWS // 16
ZLINE = CZ // 4
OUT_CH = 1000
NOUT = SEGS // OUT_CH

_mesh = plsc.VectorSubcoreMesh(core_axis_name="c", subcore_axis_name="s")


def _make_sc_scatter(with_counts, nx):
    Cc = 400 if with_counts else 640
    nchunk = E // Cc
    nouter = (-(-nchunk // 16) + 1) // 2

    out_type = [jax.ShapeDtypeStruct((N, R * F), jnp.float32)
                for _ in range(nx)]
    scratch = [
        pltpu.VMEM_SHARED((ROWS, F), jnp.float32),
        pltpu.VMEM((Cc,), jnp.int32),
        pltpu.VMEM((Cc,), jnp.int32),
        pltpu.VMEM((Cc,), jnp.int32),
        pltpu.VMEM((Cc,), jnp.int32),
        pltpu.VMEM((Cc,), jnp.int32),
        pltpu.VMEM((Cc,), jnp.int32),
        pltpu.VMEM((Cc,), jnp.int32),
        pltpu.VMEM((Cc,), jnp.int32),
        pltpu.VMEM((Cc, F), jnp.float32),
        pltpu.VMEM((Cc, F), jnp.float32),
        pltpu.SemaphoreType.DMA,
        pltpu.SemaphoreType.DMA,
        pltpu.SemaphoreType.DMA,
        pltpu.SemaphoreType.DMA,
        pltpu.SemaphoreType.DMA,
        pltpu.SemaphoreType.DMA,
    ]
    if with_counts:
        out_type.append(jax.ShapeDtypeStruct((R * N,), jnp.float32))
        scratch += [
            pltpu.VMEM_SHARED((CROWS,), jnp.float32),
            pltpu.VMEM((Cc,), jnp.int32),
            pltpu.VMEM((Cc,), jnp.int32),
            pltpu.VMEM((Cc,), jnp.float32),
            pltpu.VMEM((ZLINE,), jnp.float32),
            pltpu.SemaphoreType.DMA,
            pltpu.SemaphoreType.DMA,
        ]

    def body(*refs):
        xs = refs[:nx]
        ei_hbm, et_hbm = refs[nx:nx + 2]
        aggrs = refs[nx + 2:2 * nx + 2]
        rest = refs[2 * nx + 2:]
        if with_counts:
            (cnt_hbm, shared, src0, src1, dst0, dst1, typ0, typ1,
             seg0, seg1, xb0, xb1, is0, is1, gs0, gs1, ss0, ss1,
             cshared, sgc0, sgc1, onesv, zflat, cs0, cs1) = rest
            segcv = (sgc0, sgc1)
            csem = (cs0, cs1)
        else:
            (shared, src0, src1, dst0, dst1, typ0, typ1,
             seg0, seg1, xb0, xb1, is0, is1, gs0, gs1, ss0, ss1) = rest
        srcv = (src0, src1)
        dstv = (dst0, dst1)
        typv = (typ0, typ1)
        segv = (seg0, seg1)
        xbuf = (xb0, xb1)
        isem = (is0, is1)
        gsem = (gs0, gs1)
        ssem = (ss0, ss1)
        cc = lax.axis_index("c")
        s = lax.axis_index("s")
        base = cc * NH
        dump = SEGS + s

        def idx_copies(cid, p):
            e0 = cid * Cc
            return (
                pltpu.make_async_copy(ei_hbm.at[0, pl.ds(e0, Cc)],
                                      srcv[p], isem[p]),
                pltpu.make_async_copy(ei_hbm.at[1, pl.ds(e0, Cc)],
                                      dstv[p], isem[p]),
                pltpu.make_async_copy(et_hbm.at[pl.ds(e0, Cc)],
                                      typv[p], isem[p]),
            )

        def idx_start(cid, p):
            for d in idx_copies(cid, p):
                d.start()

        def idx_wait(cid, p):
            for d in idx_copies(cid, p):
                d.wait()

        def zero_accumulators(first):
            def zb(i, _):
                xb0[i, :] = jnp.zeros((F,), jnp.float32)
                return 0
            lax.fori_loop(0, Cc, zb, 0)
            off = s * ZR
            zcopies = [pltpu.make_async_copy(
                xb0, shared.at[pl.ds(off + kk * Cc, Cc)], gsem[0])
                for kk in range(ZR // Cc)]
            rem = ZR % Cc
            if rem:
                zcopies.append(pltpu.make_async_copy(
                    xb0.at[pl.ds(0, rem)],
                    shared.at[pl.ds(off + (ZR // Cc) * Cc, rem)], gsem[0]))
            for d in zcopies:
                d.start()
            if with_counts and first:
                def zf(i, _):
                    zflat[pl.ds(i * 16, 16)] = jnp.zeros((16,), jnp.float32)
                    return 0
                lax.fori_loop(0, ZLINE // 16, zf, 0)
                ccopies = [pltpu.make_async_copy(
                    zflat, cshared.at[pl.ds(s * CZ + kk * ZLINE, ZLINE)],
                    gsem[1]) for kk in range(4)]
                for d in ccopies:
                    d.start()

                def ob(i, _):
                    onesv[pl.ds(i * 16, 16)] = jnp.ones((16,), jnp.float32)
                    return 0
                lax.fori_loop(0, Cc // 16, ob, 0)
                for d in ccopies:
                    d.wait()
            for d in zcopies:
                d.wait()

        def scat_desc(p):
            return pltpu.make_async_copy(xbuf[p], shared.at[segv[p]],
                                         ssem[p])

        def cscat_desc(p):
            return pltpu.make_async_copy(onesv, cshared.at[segcv[p]],
                                         csem[p])

        def sweep(x_hbm, counts):
            def outer(i, _):
                for p in (0, 1):
                    k = 2 * i + p
                    cid = k * 16 + s

                    @pl.when(cid < nchunk)
                    def _():
                        @pl.when(i >= 1)
                        def _():
                            scat_desc(p).wait()
                            if counts:
                                cscat_desc(p).wait()
                        idx_wait(cid, p)
                        g = pltpu.async_copy(x_hbm.at[srcv[p]], xbuf[p],
                                             gsem[p])

                        def segb(j, _2):
                            d = dstv[p][pl.ds(j * 16, 16)]
                            t = typv[p][pl.ds(j * 16, 16)]
                            loc = d - base
                            ok = (loc >= 0) & (loc < NH)
                            segv[p][pl.ds(j * 16, 16)] = jnp.where(
                                ok, t * NH + loc, dump)
                            if counts:
                                segcv[p][pl.ds(j * 16, 16)] = jnp.where(
                                    ok, loc * R + t, dump)
                            return 0
                        lax.fori_loop(0, Cc // 16, segb, 0)
                        g.wait()

                        @pl.when(cid + 32 < nchunk)
                        def _():
                            idx_start(cid + 32, p)
                        pltpu.async_copy(xbuf[p], shared.at[segv[p]],
                                         ssem[p], add=True)
                        if counts:
                            pltpu.async_copy(onesv, cshared.at[segcv[p]],
                                             csem[p], add=True)
                return 0
            lax.fori_loop(0, nouter, outer, 0)

            ntile = (nchunk - s + 15) // 16
            for p in (0, 1):
                @pl.when(ntile >= p + 1)
                def _():
                    scat_desc(p).wait()
                    if counts:
                        cscat_desc(p).wait()

        def copy_out(aggr_hbm):
            def cp_desc(cid):
                r = cid // (NOUT // R)
                i0 = (cid % (NOUT // R)) * OUT_CH
                return pltpu.make_async_copy(
                    shared.at[pl.ds(r * NH + i0, OUT_CH)],
                    aggr_hbm.at[pl.ds(base + i0, OUT_CH), pl.ds(r * F, F)],
                    gsem[0])

            def cpout(k, _):
                cid = k * 16 + s

                @pl.when(cid < NOUT)
                def _():
                    cp_desc(cid).start()
                return 0
            lax.fori_loop(0, -(-NOUT // 16), cpout, 0)

            def cpwait(k, _):
                cid = k * 16 + s

                @pl.when(cid < NOUT)
                def _():
                    cp_desc(cid).wait()
                return 0
            lax.fori_loop(0, -(-NOUT // 16), cpwait, 0)

        for xi in range(nx):
            counts = with_counts and xi == 0
            idx_start(s, 0)
            idx_start(16 + s, 1)
            zero_accumulators(first=xi == 0)
            plsc.subcore_barrier()
            sweep(xs[xi], counts)
            plsc.subcore_barrier()
            copy_out(aggrs[xi])
            if counts:
                def cc_desc(cid):
                    return pltpu.make_async_copy(
                        cshared.at[pl.ds(cid * OUT_CH, OUT_CH)],
                        cnt_hbm.at[pl.ds(cc * SEGS + cid * OUT_CH, OUT_CH)],
                        gsem[1])

                def cpc(k, _):
                    cid = k * 16 + s

                    @pl.when(cid < NOUT)
                    def _():
                        cc_desc(cid).start()
                    return 0
                lax.fori_loop(0, -(-NOUT // 16), cpc, 0)

                def cpcw(k, _):
                    cid = k * 16 + s

                    @pl.when(cid < NOUT)
                    def _():
                        cc_desc(cid).wait()
                    return 0
                lax.fori_loop(0, -(-NOUT // 16), cpcw, 0)
            if xi + 1 < nx:
                plsc.subcore_barrier()

    return pl.kernel(body, out_type=tuple(out_type) if len(out_type) > 1
                     else out_type[0],
                     mesh=_mesh, scratch_types=scratch,
                     compiler_params=pltpu.CompilerParams(
                         use_tc_tiling_on_sc=False))


_sc_scatter_counts = _make_sc_scatter(True, 1)
_sc_scatter = _make_sc_scatter(False, 1)
_sc_scatter2 = _make_sc_scatter(False, 2)


def _dense(h_parts, aggrs, cnt2, root, W, b, fout, relu, split):
    B = 2000
    P = len(h_parts)
    fin = P * F
    in_specs = (
        [pl.BlockSpec((B, F), lambda i: (i, 0)) for _ in range(P)]
        + [pl.BlockSpec((B, R * F), lambda i: (i, 0)) for _ in range(P)]
        + [pl.BlockSpec((B, R), lambda i: (i, 0)),
           pl.BlockSpec((fin, fout), lambda i: (0, 0)),
           pl.BlockSpec((R, fin, fout), lambda i: (0, 0, 0)),
           pl.BlockSpec((1, fout), lambda i: (0, 0))]
    )
    if split:
        out_shape = [jax.ShapeDtypeStruct((N, F), jnp.float32)] * 2
        out_specs = [pl.BlockSpec((B, F), lambda i: (i, 0))] * 2
    else:
        out_shape = jax.ShapeDtypeStruct((N, fout), jnp.float32)
        out_specs = pl.BlockSpec((B, fout), lambda i: (i, 0))

    def body(*refs):
        hs = refs[:P]
        ags = refs[P:2 * P]
        cref, rref, wref, bref = refs[2 * P:2 * P + 4]
        outs = refs[2 * P + 4:]
        inv = 1.0 / jnp.maximum(cref[...], 1.0)
        sel = jnp.repeat(jnp.eye(R, dtype=jnp.float32), F, axis=1)
        scale = jnp.dot(inv, sel, preferred_element_type=jnp.float32)
        parts = [h[...] for h in hs]
        for aref in ags:
            parts.append(aref[...] * scale)
        xcat = jnp.concatenate(parts, axis=-1)
        rv = rref[...]
        wv = wref[...]
        wparts = [rv[p * F:(p + 1) * F] for p in range(P)]
        for p in range(P):
            for r in range(R):
                wparts.append(wv[r][p * F:(p + 1) * F])
        wcat = jnp.concatenate(wparts, axis=0)
        y = jnp.dot(xcat, wcat, preferred_element_type=jnp.float32)
        y = y + bref[...]
        if relu:
            y = jnp.maximum(y, 0.0)
        if split:
            outs[0][...] = y[:, :F]
            outs[1][...] = y[:, F:]
        else:
            outs[0][...] = y

    return pl.pallas_call(body, grid=(N // B,), in_specs=in_specs,
                          out_specs=out_specs, out_shape=out_shape)(
        *h_parts, *aggrs, cnt2, root, W, b)


def kernel(x, edge_index, edge_type, W1, root1, b1, W2, root2, b2, W3, root3, b3):
    ei = edge_index
    et = edge_type

    aggr1, cntf = _sc_scatter_counts(x, ei, et)
    cnt2 = cntf.reshape(N, R)

    h2 = _dense([x], [aggr1], cnt2, root1, W1, b1.reshape(1, -1), 16,
                relu=True, split=False)

    aggr2 = _sc_scatter(h2, ei, et)
    h3a, h3b = _dense([h2], [aggr2], cnt2, root2, W2, b2.reshape(1, -1), 32,
                      relu=True, split=True)

    aggr3a, aggr3b = _sc_scatter2(h3a, h3b, ei, et)
    out = _dense([h3a, h3b], [aggr3a, aggr3b],
                 cnt2, root3, W3, b3.reshape(1, -1), 64,
                 relu=False, split=False)
    return out
